# baseline (speedup 1.0000x reference)
import functools
import jax
import jax.numpy as jnp
from jax import lax
from jax.experimental import pallas as pl
from jax.experimental.pallas import tpu as pltpu
from jax.experimental.pallas import tpu_sc as plsc

EMBED_DIM = 64
RANK = 16
LANES = 16
NUM_CORES = 2
NUM_SUBCORES = 16
NUM_WORKERS = NUM_CORES * NUM_SUBCORES
BRPC = 4
GROUP = 5
DBLK = EMBED_DIM // LANES


def _lora_embed(idx, table, A, M, batch, hist):
    br_per_worker = batch // NUM_WORKERS
    num_chunks = br_per_worker // BRPC
    mesh = plsc.VectorSubcoreMesh(core_axis_name="c", subcore_axis_name="s")

    @functools.partial(
        pl.kernel,
        mesh=mesh,
        compiler_params=pltpu.CompilerParams(use_tc_tiling_on_sc=False),
        out_type=jax.ShapeDtypeStruct((batch, hist, EMBED_DIM), jnp.float32),
        scratch_types=[
            pltpu.VMEM((2, BRPC, hist), jnp.int32),
            pltpu.VMEM((2, BRPC, hist, EMBED_DIM), jnp.float32),
            pltpu.VMEM((2, BRPC, hist, RANK), jnp.float32),
            pltpu.VMEM((2, BRPC, hist, EMBED_DIM), jnp.float32),
            pltpu.VMEM((RANK, EMBED_DIM), jnp.float32),
            pltpu.SemaphoreType.DMA((2,)),
            pltpu.SemaphoreType.DMA((2,)),
            pltpu.SemaphoreType.DMA((2,)),
        ],
    )
    def kern(idx_hbm, t_hbm, a_hbm, m_hbm, out_hbm,
             slab_v, trows_v, arows_v, obuf_v, m_v, isem, gsem, osem):
        wid = lax.axis_index("s") * NUM_CORES + lax.axis_index("c")
        br0 = wid * br_per_worker
        pltpu.sync_copy(m_hbm, m_v)

        def slab_copy(c, b):
            return pltpu.make_async_copy(
                idx_hbm.at[pl.ds(br0 + c * BRPC, BRPC)], slab_v.at[b],
                isem.at[b])

        def gathers(b, r):
            return (
                pltpu.make_async_copy(
                    t_hbm.at[slab_v.at[b, r]], trows_v.at[b, r], gsem.at[b]),
                pltpu.make_async_copy(
                    a_hbm.at[slab_v.at[b, r]], arows_v.at[b, r], gsem.at[b]),
            )

        def out_copy(c, b):
            return pltpu.make_async_copy(
                obuf_v.at[b], out_hbm.at[pl.ds(br0 + c * BRPC, BRPC)],
                osem.at[b])

        slab_copy(0, 0).start()

        def chunk_body(c, _):
            b = lax.rem(c, 2)
            slab_copy(c, b).wait()
            for r in range(BRPC):
                for cp in gathers(b, r):
                    cp.start()

            @pl.when(c + 1 < num_chunks)
            def _():
                slab_copy(c + 1, 1 - b).start()

            for r in range(BRPC):
                for cp in gathers(b, r):
                    cp.wait()

            @pl.when(c >= 2)
            def _():
                out_copy(c - 2, b).wait()

            for r in range(BRPC):
                def group_body(g, _):
                    l0 = g * GROUP
                    accs = [
                        [trows_v[b, r, l0 + t, pl.ds(k * LANES, LANES)]
                         for k in range(DBLK)]
                        for t in range(GROUP)
                    ]
                    a_rows = [arows_v[b, r, l0 + t, :] for t in range(GROUP)]
                    for rk in range(RANK):
                        m_vecs = [m_v[rk, pl.ds(k * LANES, LANES)]
                                  for k in range(DBLK)]
                        for t in range(GROUP):
                            s = a_rows[t][rk]
                            for k in range(DBLK):
                                accs[t][k] = accs[t][k] + s * m_vecs[k]
                    for t in range(GROUP):
                        for k in range(DBLK):
                            obuf_v[b, r, l0 + t, pl.ds(k * LANES, LANES)] = (
                                accs[t][k])
                    return 0

                lax.fori_loop(0, hist // GROUP, group_body, 0)

            out_copy(c, b).start()
            return 0

        lax.fori_loop(0, num_chunks, chunk_body, 0)
        out_copy(num_chunks - 2, 0).wait()
        out_copy(num_chunks - 1, 1).wait()

    return kern(idx, table, A, M)


def kernel(input, table, A, B_w, C_w):
    B, L = input.shape
    M = B_w.T @ C_w.T
    return _lora_embed(input.astype(jnp.int32), table, A, M, B, L)
